# baseline (speedup 1.0000x reference)
import jax
import jax.numpy as jnp
from jax import lax
from jax.experimental import pallas as pl
from jax.experimental.pallas import tpu as pltpu
from jax.experimental.pallas import tpu_sc as plsc

BATCH = 16384
FIELDS = 26
NUM_FEATURES = 32
TOTAL = BATCH * FIELDS
PADF = 128

NUM_CORES = 2
NUM_SUBCORES = 16
NW = NUM_CORES * NUM_SUBCORES

LANES = 128
BTILES = BATCH // LANES
BTILES_PER_W = BTILES // NW

NCHUNKS = FIELDS * BTILES
CHUNKS_PER_W = NCHUNKS // NW
NPAIRS = CHUNKS_PER_W // 2


def _ids_body(idsT_hbm, flat_hbm, buf_v):
    c = lax.axis_index("c")
    s = lax.axis_index("s")
    wid = s * NUM_CORES + c
    for t in range(BTILES_PER_W):
        bt = wid * BTILES_PER_W + t
        b0 = bt * LANES
        pltpu.sync_copy(idsT_hbm.at[:, pl.ds(b0, LANES)], buf_v)
        for f in range(FIELDS):
            pltpu.sync_copy(buf_v.at[f], flat_hbm.at[pl.ds(f * BATCH + b0, LANES)])


def _gather_body(flat_hbm, table_hbm, outT_hbm, idx_vs, rows_vs, trans_vs, gsems, wsems):
    c = lax.axis_index("c")
    s = lax.axis_index("s")
    wid = s * NUM_CORES + c
    base = wid * CHUNKS_PER_W

    def stage(t, buf):
        chunk = base + lax.rem(t, CHUNKS_PER_W)
        f = chunk // BTILES
        b0 = lax.rem(chunk, BTILES) * LANES
        pltpu.sync_copy(flat_hbm.at[pl.ds(f * BATCH + b0, LANES)], idx_vs[buf])
        return pltpu.async_copy(table_hbm.at[idx_vs[buf]], rows_vs[buf], gsems[buf])

    def transpose(buf):
        rows = rows_vs[buf]
        trans = trans_vs[buf]
        lane = jnp.arange(16, dtype=jnp.int32)
        for d in range(NUM_FEATURES):
            didx = jnp.full((16,), d, dtype=jnp.int32)
            for g in range(8):
                bidx = lane + (g * 16)
                v = plsc.load_gather(rows, [bidx, didx])
                trans[d, pl.ds(g * 16, 16)] = v

    def write(t, buf):
        chunk = base + t
        f = chunk // BTILES
        b0 = lax.rem(chunk, BTILES) * LANES
        descs = []
        for k in range(4):
            descs.append(
                pltpu.async_copy(
                    trans_vs[buf].at[pl.ds(k * 8, 8), :],
                    outT_hbm.at[f, pl.ds(k * 8, 8), pl.ds(b0, LANES)],
                    wsems[buf],
                )
            )
        return descs

    def wait_gather(buf):
        pltpu.make_async_copy(table_hbm.at[pl.ds(0, LANES)], rows_vs[buf], gsems[buf]).wait()

    def wait_writes(buf):
        pltpu.make_async_copy(
            outT_hbm.at[0, :, pl.ds(0, LANES)], trans_vs[buf], wsems[buf]
        ).wait()

    def process(t, buf, wait_write):
        if wait_write:
            wait_writes(buf)
        wait_gather(buf)
        transpose(buf)
        write(t, buf)
        stage(t + 2, buf)

    stage(0, 0)
    stage(1, 1)
    process(0, 0, False)
    process(1, 1, False)

    def pair_body(j, carry):
        t0 = j * 2
        process(t0, 0, True)
        process(t0 + 1, 1, True)
        return carry

    lax.fori_loop(1, NPAIRS, pair_body, 0)

    for buf in range(2):
        wait_gather(buf)
        wait_writes(buf)


@jax.jit
def kernel(test_feature_ids, feature_vectors):
    mesh = plsc.VectorSubcoreMesh(core_axis_name="c", subcore_axis_name="s")
    flat_ids = pl.kernel(
        _ids_body,
        out_type=jax.ShapeDtypeStruct((TOTAL,), jnp.int32),
        mesh=mesh,
        scratch_types=[pltpu.VMEM((FIELDS, LANES), jnp.int32)],
        compiler_params=pltpu.CompilerParams(use_tc_tiling_on_sc=True, needs_layout_passes=False),
    )(test_feature_ids.T)
    padded_table = jnp.pad(feature_vectors, ((0, 0), (0, PADF - NUM_FEATURES)))
    outT = pl.kernel(
        _gather_body,
        out_type=jax.ShapeDtypeStruct((FIELDS, NUM_FEATURES, BATCH), jnp.float32),
        mesh=mesh,
        scratch_types=[
            [pltpu.VMEM((LANES,), jnp.int32)] * 2,
            [pltpu.VMEM((LANES, PADF), jnp.float32)] * 2,
            [pltpu.VMEM((NUM_FEATURES, LANES), jnp.float32)] * 2,
            [pltpu.SemaphoreType.DMA] * 2,
            [pltpu.SemaphoreType.DMA] * 2,
        ],
        compiler_params=pltpu.CompilerParams(use_tc_tiling_on_sc=True, needs_layout_passes=False),
    )(flat_ids, padded_table)
    return outT.transpose(2, 0, 1)


# speedup vs baseline: 1.0204x; 1.0204x over previous
import jax
import jax.numpy as jnp
from jax import lax
from jax.experimental import pallas as pl
from jax.experimental.pallas import tpu as pltpu
from jax.experimental.pallas import tpu_sc as plsc

BATCH = 16384
FIELDS = 26
NUM_FEATURES = 32
TOTAL = BATCH * FIELDS
PADF = 128

NUM_CORES = 2
NUM_SUBCORES = 16
NW = NUM_CORES * NUM_SUBCORES

LANES = 128
BTILES = BATCH // LANES
BTILES_PER_W = BTILES // NW

ROWS_PER_W = TOTAL // NW
CHUNK = 256
CHUNKS_PER_W = ROWS_PER_W // CHUNK
NPAIRS = CHUNKS_PER_W // 2


def _ids_body(idsT_hbm, flat_hbm, buf_v):
    c = lax.axis_index("c")
    s = lax.axis_index("s")
    wid = s * NUM_CORES + c
    for t in range(BTILES_PER_W):
        bt = wid * BTILES_PER_W + t
        b0 = bt * LANES
        pltpu.sync_copy(idsT_hbm.at[:, pl.ds(b0, LANES)], buf_v)
        for f in range(FIELDS):
            pltpu.sync_copy(buf_v.at[f], flat_hbm.at[pl.ds(f * BATCH + b0, LANES)])


def _gather_body(flat_hbm, table_hbm, outT_hbm, idx_all, rows_vs, trans_vs, gsems, wsems):
    c = lax.axis_index("c")
    s = lax.axis_index("s")
    wid = s * NUM_CORES + c
    r_base = wid * ROWS_PER_W

    pltpu.sync_copy(flat_hbm.at[pl.ds(r_base, ROWS_PER_W)], idx_all)

    def stage(t, buf):
        off = lax.rem(t, CHUNKS_PER_W) * CHUNK
        return pltpu.async_copy(
            table_hbm.at[idx_all.at[pl.ds(off, CHUNK)]], rows_vs[buf], gsems[buf]
        )

    def transpose(buf):
        rows = rows_vs[buf]
        trans = trans_vs[buf]
        lane = jnp.arange(16, dtype=jnp.int32)
        for d in range(NUM_FEATURES):
            didx = jnp.full((16,), d, dtype=jnp.int32)
            for g in range(CHUNK // 16):
                bidx = lane + (g * 16)
                v = plsc.load_gather(rows, [bidx, didx])
                trans[d, pl.ds(g * 16, 16)] = v

    def write(t, buf):
        r0 = r_base + t * CHUNK
        f = r0 // BATCH
        b0 = lax.rem(r0, BATCH)
        descs = []
        for k in range(4):
            for h in range(CHUNK // LANES):
                descs.append(
                    pltpu.async_copy(
                        trans_vs[buf].at[pl.ds(k * 8, 8), pl.ds(h * LANES, LANES)],
                        outT_hbm.at[f, pl.ds(k * 8, 8), pl.ds(b0 + h * LANES, LANES)],
                        wsems[buf],
                    )
                )
        return descs

    def wait_gather(buf):
        pltpu.make_async_copy(table_hbm.at[pl.ds(0, CHUNK)], rows_vs[buf], gsems[buf]).wait()

    def wait_writes(buf):
        pltpu.make_async_copy(
            outT_hbm.at[0, :, pl.ds(0, CHUNK)], trans_vs[buf], wsems[buf]
        ).wait()

    def process(t, buf, wait_write):
        if wait_write:
            wait_writes(buf)
        wait_gather(buf)
        transpose(buf)
        write(t, buf)
        stage(t + 2, buf)

    stage(0, 0)
    stage(1, 1)
    process(0, 0, False)
    process(1, 1, False)

    def pair_body(j, carry):
        t0 = j * 2
        process(t0, 0, True)
        process(t0 + 1, 1, True)
        return carry

    lax.fori_loop(1, NPAIRS, pair_body, 0)

    for buf in range(2):
        wait_gather(buf)
        wait_writes(buf)


@jax.jit
def kernel(test_feature_ids, feature_vectors):
    mesh = plsc.VectorSubcoreMesh(core_axis_name="c", subcore_axis_name="s")
    flat_ids = pl.kernel(
        _ids_body,
        out_type=jax.ShapeDtypeStruct((TOTAL,), jnp.int32),
        mesh=mesh,
        scratch_types=[pltpu.VMEM((FIELDS, LANES), jnp.int32)],
        compiler_params=pltpu.CompilerParams(use_tc_tiling_on_sc=True, needs_layout_passes=False),
    )(test_feature_ids.T)
    padded_table = jnp.pad(feature_vectors, ((0, 0), (0, PADF - NUM_FEATURES)))
    outT = pl.kernel(
        _gather_body,
        out_type=jax.ShapeDtypeStruct((FIELDS, NUM_FEATURES, BATCH), jnp.float32),
        mesh=mesh,
        scratch_types=[
            pltpu.VMEM((ROWS_PER_W,), jnp.int32),
            [pltpu.VMEM((CHUNK, PADF), jnp.float32)] * 2,
            [pltpu.VMEM((NUM_FEATURES, CHUNK), jnp.float32)] * 2,
            [pltpu.SemaphoreType.DMA] * 2,
            [pltpu.SemaphoreType.DMA] * 2,
        ],
        compiler_params=pltpu.CompilerParams(use_tc_tiling_on_sc=True, needs_layout_passes=False),
    )(flat_ids, padded_table)
    return outT.transpose(2, 0, 1)


# speedup vs baseline: 1.0415x; 1.0207x over previous
import jax
import jax.numpy as jnp
from jax import lax
from jax.experimental import pallas as pl
from jax.experimental.pallas import tpu as pltpu
from jax.experimental.pallas import tpu_sc as plsc

BATCH = 16384
FIELDS = 26
NUM_FEATURES = 32
TOTAL = BATCH * FIELDS
PADF = 128

NUM_CORES = 2
NUM_SUBCORES = 16
NW = NUM_CORES * NUM_SUBCORES

LANES = 128
BTILES = BATCH // LANES
BTILES_PER_W = BTILES // NW

ROWS_PER_W = TOTAL // NW
CHUNK = 256
CHUNKS_PER_W = ROWS_PER_W // CHUNK
NPAIRS = CHUNKS_PER_W // 2


def _ids_body(idsT_hbm, flat_hbm, buf_v):
    c = lax.axis_index("c")
    s = lax.axis_index("s")
    wid = s * NUM_CORES + c
    for t in range(BTILES_PER_W):
        bt = wid * BTILES_PER_W + t
        b0 = bt * LANES
        pltpu.sync_copy(idsT_hbm.at[:, pl.ds(b0, LANES)], buf_v)
        for f in range(FIELDS):
            for g in range(LANES // 16):
                sl = pl.ds(g * 16, 16)
                buf_v[f, sl] = buf_v[f, sl] * 4
            pltpu.sync_copy(buf_v.at[f], flat_hbm.at[pl.ds(f * BATCH + b0, LANES)])


def _gather_body(flat_hbm, table_hbm, out_hbm, idx_all, rows_vs, trans_vs, gsems, wsems):
    c = lax.axis_index("c")
    s = lax.axis_index("s")
    wid = s * NUM_CORES + c
    r_base = wid * ROWS_PER_W

    pltpu.sync_copy(flat_hbm.at[pl.ds(r_base, ROWS_PER_W)], idx_all)

    def stage(t, buf):
        off = lax.rem(t, CHUNKS_PER_W) * CHUNK
        return pltpu.async_copy(
            table_hbm.at[idx_all.at[pl.ds(off, CHUNK)]], rows_vs[buf], gsems[buf]
        )

    def transpose(buf):
        rows = rows_vs[buf]
        trans = trans_vs[buf]
        lane = jnp.arange(16, dtype=jnp.int32)
        for d in range(NUM_FEATURES):
            didx = jnp.full((16,), d, dtype=jnp.int32)
            for g in range(CHUNK // 16):
                bidx = lane + (g * 16)
                v = plsc.load_gather(rows, [bidx, didx])
                trans[d, pl.ds(g * 16, 16)] = v

    def write(t, buf):
        r0 = r_base + t * CHUNK
        f = r0 // BATCH
        bt0 = lax.rem(r0, BATCH) // LANES
        descs = []
        for k in range(4):
            for h in range(CHUNK // LANES):
                descs.append(
                    pltpu.async_copy(
                        trans_vs[buf].at[pl.ds(k * 8, 8), pl.ds(h * LANES, LANES)],
                        out_hbm.at[f, k, bt0 + h],
                        wsems[buf],
                    )
                )
        return descs

    def wait_gather(buf):
        pltpu.make_async_copy(
            table_hbm.at[pl.ds(0, CHUNK)], rows_vs[buf], gsems[buf]
        ).wait()

    def wait_writes(buf):
        pltpu.make_async_copy(
            table_hbm.at[pl.ds(0, CHUNK)], rows_vs[buf], wsems[buf]
        ).wait()

    def process(t, buf, wait_write):
        if wait_write:
            wait_writes(buf)
        wait_gather(buf)
        transpose(buf)
        write(t, buf)
        stage(t + 2, buf)

    stage(0, 0)
    stage(1, 1)
    process(0, 0, False)
    process(1, 1, False)

    def pair_body(j, carry):
        t0 = j * 2
        process(t0, 0, True)
        process(t0 + 1, 1, True)
        return carry

    lax.fori_loop(1, NPAIRS, pair_body, 0)

    for buf in range(2):
        wait_gather(buf)
        wait_writes(buf)


@jax.jit
def kernel(test_feature_ids, feature_vectors):
    mesh = plsc.VectorSubcoreMesh(core_axis_name="c", subcore_axis_name="s")
    flat_ids = pl.kernel(
        _ids_body,
        out_type=jax.ShapeDtypeStruct((TOTAL,), jnp.int32),
        mesh=mesh,
        scratch_types=[pltpu.VMEM((FIELDS, LANES), jnp.int32)],
        compiler_params=pltpu.CompilerParams(
            use_tc_tiling_on_sc=True, needs_layout_passes=False
        ),
    )(test_feature_ids.T)
    padded_table = jnp.pad(feature_vectors, ((0, 0), (0, PADF - NUM_FEATURES)))
    tview = padded_table.reshape(4 * 1000000, NUM_FEATURES)
    out5d = pl.kernel(
        _gather_body,
        out_type=jax.ShapeDtypeStruct(
            (FIELDS, 4, BTILES, 8, LANES), jnp.float32
        ),
        mesh=mesh,
        scratch_types=[
            pltpu.VMEM((ROWS_PER_W,), jnp.int32),
            [pltpu.VMEM((CHUNK, NUM_FEATURES), jnp.float32)] * 2,
            [pltpu.VMEM((NUM_FEATURES, CHUNK), jnp.float32)] * 2,
            [pltpu.SemaphoreType.DMA] * 2,
            [pltpu.SemaphoreType.DMA] * 2,
        ],
        compiler_params=pltpu.CompilerParams(
            use_tc_tiling_on_sc=False, needs_layout_passes=False
        ),
    )(flat_ids, tview)
    return out5d.transpose(2, 4, 0, 1, 3).reshape(BATCH, FIELDS, NUM_FEATURES)


# speedup vs baseline: 1.1451x; 1.0995x over previous
import jax
import jax.numpy as jnp
from jax import lax
from jax.experimental import pallas as pl
from jax.experimental.pallas import tpu as pltpu
from jax.experimental.pallas import tpu_sc as plsc

BATCH = 16384
FIELDS = 26
NUM_FEATURES = 32
TOTAL = BATCH * FIELDS
PADF = 128

NUM_CORES = 2
NUM_SUBCORES = 16
NW = NUM_CORES * NUM_SUBCORES

LANES = 128
BTILES = BATCH // LANES
BTILES_PER_W = BTILES // NW

ROWS_PER_W = TOTAL // NW
CHUNK = 256
CHUNKS_PER_W = ROWS_PER_W // CHUNK
NPAIRS = CHUNKS_PER_W // 2


def _ids_body(idsT_hbm, flat_hbm, buf_v):
    c = lax.axis_index("c")
    s = lax.axis_index("s")
    wid = s * NUM_CORES + c
    for t in range(BTILES_PER_W):
        bt = wid * BTILES_PER_W + t
        b0 = bt * LANES
        pltpu.sync_copy(idsT_hbm.at[:, pl.ds(b0, LANES)], buf_v)
        for f in range(FIELDS):
            for g in range(LANES // 16):
                sl = pl.ds(g * 16, 16)
                buf_v[f, sl] = buf_v[f, sl] * 4
            pltpu.sync_copy(buf_v.at[f], flat_hbm.at[pl.ds(f * BATCH + b0, LANES)])


def _gather_body(flat_hbm, table_hbm, out_hbm, idx_all, rows_vs, trans_vs, gsems, wsems):
    c = lax.axis_index("c")
    s = lax.axis_index("s")
    wid = s * NUM_CORES + c
    r_base = wid * ROWS_PER_W

    pltpu.sync_copy(flat_hbm.at[pl.ds(r_base, ROWS_PER_W)], idx_all)

    def stage(t, buf):
        off = lax.rem(t, CHUNKS_PER_W) * CHUNK
        return pltpu.async_copy(
            table_hbm.at[idx_all.at[pl.ds(off, CHUNK)]], rows_vs[buf], gsems[buf]
        )

    def transpose(buf):
        rows = rows_vs[buf]
        trans = trans_vs[buf]
        lane = jnp.arange(16, dtype=jnp.int32)
        for d in range(NUM_FEATURES):
            didx = jnp.full((16,), d, dtype=jnp.int32)
            for g0 in range(0, CHUNK // 16, 4):
                vs = [
                    plsc.load_gather(rows, [lane + (g * 16), didx])
                    for g in range(g0, g0 + 4)
                ]
                for i, g in enumerate(range(g0, g0 + 4)):
                    trans[d, pl.ds(g * 16, 16)] = vs[i]

    def write(t, buf):
        r0 = r_base + t * CHUNK
        f = r0 // BATCH
        bt0 = lax.rem(r0, BATCH) // LANES
        descs = []
        for k in range(4):
            for h in range(CHUNK // LANES):
                descs.append(
                    pltpu.async_copy(
                        trans_vs[buf].at[pl.ds(k * 8, 8), pl.ds(h * LANES, LANES)],
                        out_hbm.at[f, k, bt0 + h],
                        wsems[buf],
                    )
                )
        return descs

    def wait_gather(buf):
        pltpu.make_async_copy(
            table_hbm.at[pl.ds(0, CHUNK)], rows_vs[buf], gsems[buf]
        ).wait()

    def wait_writes(buf):
        pltpu.make_async_copy(
            table_hbm.at[pl.ds(0, CHUNK)], rows_vs[buf], wsems[buf]
        ).wait()

    def process(t, buf, wait_write):
        if wait_write:
            wait_writes(buf)
        wait_gather(buf)
        transpose(buf)
        write(t, buf)
        stage(t + 2, buf)

    stage(0, 0)
    stage(1, 1)
    process(0, 0, False)
    process(1, 1, False)

    def pair_body(j, carry):
        t0 = j * 2
        process(t0, 0, True)
        process(t0 + 1, 1, True)
        return carry

    lax.fori_loop(1, NPAIRS, pair_body, 0)

    for buf in range(2):
        wait_gather(buf)
        wait_writes(buf)


@jax.jit
def kernel(test_feature_ids, feature_vectors):
    mesh = plsc.VectorSubcoreMesh(core_axis_name="c", subcore_axis_name="s")
    flat_ids = pl.kernel(
        _ids_body,
        out_type=jax.ShapeDtypeStruct((TOTAL,), jnp.int32),
        mesh=mesh,
        scratch_types=[pltpu.VMEM((FIELDS, LANES), jnp.int32)],
        compiler_params=pltpu.CompilerParams(
            use_tc_tiling_on_sc=True, needs_layout_passes=False
        ),
    )(test_feature_ids.T)
    padded_table = jnp.pad(feature_vectors, ((0, 0), (0, PADF - NUM_FEATURES)))
    tview = padded_table.reshape(4 * 1000000, NUM_FEATURES)
    out5d = pl.kernel(
        _gather_body,
        out_type=jax.ShapeDtypeStruct(
            (FIELDS, 4, BTILES, 8, LANES), jnp.float32
        ),
        mesh=mesh,
        scratch_types=[
            pltpu.VMEM((ROWS_PER_W,), jnp.int32),
            [pltpu.VMEM((CHUNK, NUM_FEATURES), jnp.float32)] * 2,
            [pltpu.VMEM((NUM_FEATURES, CHUNK), jnp.float32)] * 2,
            [pltpu.SemaphoreType.DMA] * 2,
            [pltpu.SemaphoreType.DMA] * 2,
        ],
        compiler_params=pltpu.CompilerParams(
            use_tc_tiling_on_sc=False, needs_layout_passes=False
        ),
    )(flat_ids, tview)
    return out5d.transpose(2, 4, 0, 1, 3).reshape(BATCH, FIELDS, NUM_FEATURES)


# speedup vs baseline: 1.1639x; 1.0164x over previous
import jax
import jax.numpy as jnp
from jax import lax
from jax.experimental import pallas as pl
from jax.experimental.pallas import tpu as pltpu
from jax.experimental.pallas import tpu_sc as plsc

BATCH = 16384
FIELDS = 26
NUM_FEATURES = 32
TOTAL = BATCH * FIELDS
PADF = 128

NUM_CORES = 2
NUM_SUBCORES = 16
NW = NUM_CORES * NUM_SUBCORES

LANES = 128
BTILES = BATCH // LANES
BTILES_PER_W = BTILES // NW

ROWS_PER_W = TOTAL // NW
CHUNK = 256
CHUNKS_PER_W = ROWS_PER_W // CHUNK
NPAIRS = CHUNKS_PER_W // 2


def _ids_body(idsT_hbm, flat_hbm, buf_v):
    c = lax.axis_index("c")
    s = lax.axis_index("s")
    wid = s * NUM_CORES + c
    for t in range(BTILES_PER_W):
        bt = wid * BTILES_PER_W + t
        b0 = bt * LANES
        pltpu.sync_copy(idsT_hbm.at[:, pl.ds(b0, LANES)], buf_v)
        for f in range(FIELDS):
            for g in range(LANES // 16):
                sl = pl.ds(g * 16, 16)
                buf_v[f, sl] = buf_v[f, sl] * 4
            pltpu.sync_copy(buf_v.at[f], flat_hbm.at[pl.ds(f * BATCH + b0, LANES)])


def _gather_body(flat_hbm, table_hbm, out_hbm, idx_all, rows_vs, trans_vs, gsems, wsems):
    c = lax.axis_index("c")
    s = lax.axis_index("s")
    wid = s * NUM_CORES + c
    r_base = wid * ROWS_PER_W

    pltpu.sync_copy(flat_hbm.at[pl.ds(r_base, ROWS_PER_W)], idx_all)

    def stage(t, buf):
        off = lax.rem(t, CHUNKS_PER_W) * CHUNK
        return pltpu.async_copy(
            table_hbm.at[idx_all.at[pl.ds(off, CHUNK)]], rows_vs[buf], gsems[buf]
        )

    def transpose(buf):
        rows = rows_vs[buf]
        trans = trans_vs[buf]
        lane = jnp.arange(16, dtype=jnp.int32)
        for d in range(NUM_FEATURES):
            didx = jnp.full((16,), d, dtype=jnp.int32)
            for g0 in range(0, CHUNK // 16, 8):
                vs = [
                    plsc.load_gather(rows, [lane + (g * 16), didx])
                    for g in range(g0, g0 + 8)
                ]
                for i, g in enumerate(range(g0, g0 + 8)):
                    trans[d, pl.ds(g * 16, 16)] = vs[i]

    def write(t, buf):
        r0 = r_base + t * CHUNK
        f = r0 // BATCH
        bt0 = lax.rem(r0, BATCH) // LANES
        descs = []
        for k in range(4):
            for h in range(CHUNK // LANES):
                descs.append(
                    pltpu.async_copy(
                        trans_vs[buf].at[pl.ds(k * 8, 8), pl.ds(h * LANES, LANES)],
                        out_hbm.at[f, k, bt0 + h],
                        wsems[buf],
                    )
                )
        return descs

    def wait_gather(buf):
        pltpu.make_async_copy(
            table_hbm.at[pl.ds(0, CHUNK)], rows_vs[buf], gsems[buf]
        ).wait()

    def wait_writes(buf):
        pltpu.make_async_copy(
            table_hbm.at[pl.ds(0, CHUNK)], rows_vs[buf], wsems[buf]
        ).wait()

    def process(t, buf, wait_write):
        if wait_write:
            wait_writes(buf)
        wait_gather(buf)
        transpose(buf)
        write(t, buf)
        stage(t + 2, buf)

    stage(0, 0)
    stage(1, 1)
    process(0, 0, False)
    process(1, 1, False)

    def pair_body(j, carry):
        t0 = j * 2
        process(t0, 0, True)
        process(t0 + 1, 1, True)
        return carry

    lax.fori_loop(1, NPAIRS, pair_body, 0)

    for buf in range(2):
        wait_gather(buf)
        wait_writes(buf)


@jax.jit
def kernel(test_feature_ids, feature_vectors):
    mesh = plsc.VectorSubcoreMesh(core_axis_name="c", subcore_axis_name="s")
    flat_ids = pl.kernel(
        _ids_body,
        out_type=jax.ShapeDtypeStruct((TOTAL,), jnp.int32),
        mesh=mesh,
        scratch_types=[pltpu.VMEM((FIELDS, LANES), jnp.int32)],
        compiler_params=pltpu.CompilerParams(
            use_tc_tiling_on_sc=True, needs_layout_passes=False
        ),
    )(test_feature_ids.T)
    padded_table = jnp.pad(feature_vectors, ((0, 0), (0, PADF - NUM_FEATURES)))
    tview = padded_table.reshape(4 * 1000000, NUM_FEATURES)
    out5d = pl.kernel(
        _gather_body,
        out_type=jax.ShapeDtypeStruct(
            (FIELDS, 4, BTILES, 8, LANES), jnp.float32
        ),
        mesh=mesh,
        scratch_types=[
            pltpu.VMEM((ROWS_PER_W,), jnp.int32),
            [pltpu.VMEM((CHUNK, NUM_FEATURES), jnp.float32)] * 2,
            [pltpu.VMEM((NUM_FEATURES, CHUNK), jnp.float32)] * 2,
            [pltpu.SemaphoreType.DMA] * 2,
            [pltpu.SemaphoreType.DMA] * 2,
        ],
        compiler_params=pltpu.CompilerParams(
            use_tc_tiling_on_sc=False, needs_layout_passes=False
        ),
    )(flat_ids, tview)
    return out5d.transpose(2, 4, 0, 1, 3).reshape(BATCH, FIELDS, NUM_FEATURES)
